# Initial kernel scaffold; baseline (speedup 1.0000x reference)
import jax
import jax.numpy as jnp
from jax.experimental import pallas as pl


def kernel(features, edge_index, W1, al1, ar1, Wres1, W2, al2, ar2, Wres2):
    raise NotImplementedError("write your pallas kernel here")


# speedup vs baseline: 31.4915x; 31.4915x over previous
import functools

import jax
import jax.numpy as jnp
from jax import lax
from jax.experimental import pallas as pl
from jax.experimental.pallas import tpu as pltpu
from jax.experimental.pallas import tpu_sc as plsc

N = 10000
E = 320000
D_IN = 128
HEADS1 = 4
D_H1 = 64
D1 = HEADS1 * D_H1
D_OUT = 40

ROW1 = 80
ROW2 = 48
FT2W = 48

EPS_D = 1e-10
BN_EPS = 1e-5
CLAMP = 60.0

_SC_PARAMS = pltpu.CompilerParams(use_tc_tiling_on_sc=False,
                                  needs_layout_passes=False)


def _bn_cols(x):
    mu = jnp.mean(x, axis=0, keepdims=True)
    var = jnp.mean((x - mu) ** 2, axis=0, keepdims=True)
    return (x - mu) / jnp.sqrt(var + BN_EPS)


def _dense1_body(x_ref, w1_ref, alb_ref, arb_ref, wres_ref,
                 ft_ref, atab_ref, res_ref):
    x = x_ref[...]
    bn = _bn_cols(x)
    ft = jnp.dot(bn, w1_ref[...], preferred_element_type=jnp.float32)
    a1 = jnp.dot(ft, alb_ref[...], preferred_element_type=jnp.float32)
    a2 = jnp.dot(ft, arb_ref[...], preferred_element_type=jnp.float32)
    res_ref[...] = jnp.dot(bn, wres_ref[...], preferred_element_type=jnp.float32)
    for h in range(HEADS1):
        ft_ref[h * N:(h + 1) * N, :] = ft[:, h * D_H1:(h + 1) * D_H1]
    atab_ref[:, 0, :] = jnp.transpose(a1)
    atab_ref[:, 1, :] = jnp.transpose(a2)


def _dense1(features, w1cat, alblk, arblk, wres1cat):
    return pl.pallas_call(
        _dense1_body,
        out_shape=[
            jax.ShapeDtypeStruct((HEADS1 * N, D_H1), jnp.float32),
            jax.ShapeDtypeStruct((HEADS1, 2, N), jnp.float32),
            jax.ShapeDtypeStruct((N, D1), jnp.float32),
        ],
    )(features, w1cat, alblk, arblk, wres1cat)


def _dense2_body(sc1a_ref, sc1b_ref, res1_ref, w2_ref, al2_ref, ar2_ref,
                 wres2_ref, ft2_ref, atab2_ref, res2_ref):
    cols = []
    for h in range(HEADS1):
        ref = (sc1a_ref, sc1b_ref)[h // 2]
        c = h % 2
        numer = ref[c * N:(c + 1) * N, 0:D_H1]
        denom = ref[c * N:(c + 1) * N, D_H1:D_H1 + 1]
        cols.append(numer / (denom + EPS_D))
    agg = jnp.concatenate(cols, axis=1)
    h1 = jnp.maximum(agg + res1_ref[...], 0.0)
    bn = _bn_cols(h1)
    ft2 = jnp.dot(bn, w2_ref[...], preferred_element_type=jnp.float32)
    b1 = jnp.dot(ft2, al2_ref[...], preferred_element_type=jnp.float32)
    b2 = jnp.dot(ft2, ar2_ref[...], preferred_element_type=jnp.float32)
    ft2_ref[...] = jnp.concatenate(
        [ft2, jnp.zeros((N, FT2W - D_OUT), jnp.float32)], axis=1)
    atab2_ref[0:1, :] = jnp.transpose(b1)
    atab2_ref[1:2, :] = jnp.transpose(b2)
    res2_ref[...] = jnp.dot(bn, wres2_ref[...], preferred_element_type=jnp.float32)


def _dense2(sc1a, sc1b, res1, w2, al2, ar2, wres2):
    return pl.pallas_call(
        _dense2_body,
        out_shape=[
            jax.ShapeDtypeStruct((N, FT2W), jnp.float32),
            jax.ShapeDtypeStruct((2, N), jnp.float32),
            jax.ShapeDtypeStruct((N, D_OUT), jnp.float32),
        ],
    )(sc1a, sc1b, res1, w2, al2, ar2, wres2)


def _combine_body(sc2_ref, res2_ref, out_ref):
    numer = sc2_ref[0:N, 0:D_OUT] + sc2_ref[N:2 * N, 0:D_OUT]
    denom = sc2_ref[0:N, D_OUT:D_OUT + 1] + sc2_ref[N:2 * N, D_OUT:D_OUT + 1]
    out_ref[...] = numer / (denom + EPS_D) + res2_ref[...]


def _combine(sc2, res2):
    return pl.pallas_call(
        _combine_body,
        out_shape=jax.ShapeDtypeStruct((N, D_OUT), jnp.float32),
    )(sc2, res2)


def _make_edge_kernel(dft, dreal, row, ch, ept, head_base, split_edges):
    nch = ept // ch
    zrows = 16
    mesh = plsc.VectorSubcoreMesh(core_axis_name="c", subcore_axis_name="s")
    nv = row // 16
    nf = dft // 16
    dv, dl = divmod(dreal, 16)

    @functools.partial(
        pl.kernel,
        mesh=mesh,
        compiler_params=_SC_PARAMS,
        out_type=jax.ShapeDtypeStruct((2 * N, row), jnp.float32),
        scratch_types=[
            pltpu.VMEM((ept,), jnp.int32),
            pltpu.VMEM((nch, ch), jnp.int32),
            pltpu.VMEM((2, N), jnp.float32),
            pltpu.VMEM((2, ch), jnp.int32),
            pltpu.VMEM((2, ch, dft), jnp.float32),
            pltpu.VMEM((ch, row), jnp.float32),
            pltpu.VMEM((zrows, row), jnp.float32),
            pltpu.VMEM_SHARED((N, row), jnp.float32),
            pltpu.SemaphoreType.DMA,
            pltpu.SemaphoreType.DMA,
        ],
    )
    def edge_kernel(ft_hbm, atab_hbm, src_hbm, dstm_hbm, out_hbm,
                    src_v, dstm_v, atab_v, idxb, ftb, msg, zbuf,
                    accum, sem_a, sem_b):
        c = lax.axis_index("c")
        s = lax.axis_index("s")
        if split_edges:
            ebase = (c * 16 + s) * ept
            widx = c * 16 + s
            toff = c * 0
        else:
            ebase = s * ept
            widx = s
            toff = (head_base + c) * N

        pltpu.sync_copy(src_hbm.at[pl.ds(ebase, ept)], src_v)
        pltpu.sync_copy(dstm_hbm.at[widx], dstm_v)
        if split_edges:
            pltpu.sync_copy(atab_hbm.at[0], atab_v)
        else:
            pltpu.sync_copy(atab_hbm.at[head_base + c], atab_v)

        rbase = s * 624
        nz = jnp.where(s == 15, 40, 39)

        def zr(i, carry):
            for j in range(nv):
                zbuf[i, pl.ds(j * 16, 16)] = jnp.zeros((16,), jnp.float32)
            return carry
        lax.fori_loop(0, zrows, zr, 0)

        def zc(k, carry):
            pltpu.sync_copy(zbuf, accum.at[pl.ds(rbase + k * zrows, zrows)])
            return carry
        lax.fori_loop(0, nz, zc, 0)
        plsc.subcore_barrier()

        iot = lax.iota(jnp.int32, 16)
        oh = (iot == dl).astype(jnp.float32)
        zi = jnp.zeros((16,), jnp.int32)
        oi = jnp.full((16,), 1, jnp.int32)

        def stage_idx(ci, b):
            cc = jnp.minimum(ci, nch - 1)
            for g in range(ch // 16):
                v = src_v[pl.ds(cc * ch + g * 16, 16)]
                idxb[b, pl.ds(g * 16, 16)] = v + toff

        def start(b, sem):
            pltpu.make_async_copy(ft_hbm.at[idxb.at[b]], ftb.at[b], sem).start()

        def wait(b, sem):
            pltpu.make_async_copy(ft_hbm.at[idxb.at[b]], ftb.at[b], sem).wait()

        def process(ci, b):
            wv = []
            for g in range(ch // 16):
                sl = pl.ds(g * 16, 16)
                srcs = src_v[pl.ds(ci * ch + g * 16, 16)]
                dsts = dstm_v[ci, sl]
                av = plsc.load_gather(atab_v, [zi, srcs])
                bv = plsc.load_gather(atab_v, [oi, dsts])
                e = av + bv
                e = jnp.where(e >= 0.0, e, 0.01 * e)
                e = jnp.minimum(e, CLAMP)
                wv.append(jnp.exp(e))
            for k in range(ch):
                w0 = wv[k // 16][k % 16]
                for j in range(nv):
                    if j < nf:
                        fv = ftb[b, k, pl.ds(j * 16, 16)]
                        if j == dv:
                            fv = fv + oh
                    else:
                        fv = oh
                    msg[k, pl.ds(j * 16, 16)] = fv * w0
            pltpu.sync_copy(msg, accum.at[dstm_v.at[ci]], add=True)

        stage_idx(0, 0)
        start(0, sem_a)

        def step(i2, carry):
            i = i2 * 2
            wait(0, sem_a)
            stage_idx(i + 1, 1)
            start(1, sem_b)
            process(i, 0)
            wait(1, sem_b)
            stage_idx(i + 2, 0)
            start(0, sem_a)
            process(i + 1, 1)
            return carry
        lax.fori_loop(0, nch // 2, step, 0)
        wait(0, sem_a)
        process(nch - 1, 0)

        plsc.subcore_barrier()

        def oc(k, carry):
            pltpu.sync_copy(accum.at[pl.ds(rbase + k * zrows, zrows)],
                            out_hbm.at[pl.ds(c * N + rbase + k * zrows, zrows)])
            return carry
        lax.fori_loop(0, nz, oc, 0)

    return edge_kernel


def kernel(features, edge_index, W1, al1, ar1, Wres1, W2, al2, ar2, Wres2):
    _edge1a = _make_edge_kernel(dft=64, dreal=64, row=ROW1, ch=32, ept=E // 16,
                                head_base=0, split_edges=False)
    _edge1b = _make_edge_kernel(dft=64, dreal=64, row=ROW1, ch=32, ept=E // 16,
                                head_base=2, split_edges=False)
    _edge2 = _make_edge_kernel(dft=FT2W, dreal=D_OUT, row=ROW2, ch=16,
                               ept=E // 32, split_edges=True, head_base=0)
    src = edge_index[0]
    dstm1 = edge_index[1].reshape(16, E // 16 // 32, 32)
    dstm2 = edge_index[1].reshape(32, E // 32 // 16, 16)

    w1cat = jnp.transpose(W1, (1, 0, 2)).reshape(D_IN, D1)
    wres1cat = jnp.transpose(Wres1, (1, 0, 2)).reshape(D_IN, D1)
    alblk = jnp.zeros((D1, HEADS1), jnp.float32)
    arblk = jnp.zeros((D1, HEADS1), jnp.float32)
    for h in range(HEADS1):
        alblk = alblk.at[h * D_H1:(h + 1) * D_H1, h].set(al1[h, :, 0])
        arblk = arblk.at[h * D_H1:(h + 1) * D_H1, h].set(ar1[h, :, 0])

    ft_tab, atab, res1 = _dense1(features, w1cat, alblk, arblk, wres1cat)
    sc1a = _edge1a(ft_tab, atab, src, dstm1)
    sc1b = _edge1b(ft_tab, atab, src, dstm1)
    ft2_tab, atab2, res2 = _dense2(sc1a, sc1b, res1, W2, al2, ar2, Wres2)
    sc2 = _edge2(ft2_tab, atab2.reshape(1, 2, N), src, dstm2)
    return _combine(sc2, res2)


# speedup vs baseline: 57.4286x; 1.8236x over previous
import functools

import jax
import jax.numpy as jnp
from jax import lax
from jax.experimental import pallas as pl
from jax.experimental.pallas import tpu as pltpu
from jax.experimental.pallas import tpu_sc as plsc

N = 10000
E = 320000
D_IN = 128
HEADS1 = 4
D_H1 = 64
D1 = HEADS1 * D_H1
D_OUT = 40

ROW1 = 80
ROW2 = 48
FT2W = 48

EPS_D = 1e-10
BN_EPS = 1e-5
CLAMP = 60.0

_SC_PARAMS = pltpu.CompilerParams(use_tc_tiling_on_sc=False,
                                  needs_layout_passes=False)


def _bn_cols(x):
    mu = jnp.mean(x, axis=0, keepdims=True)
    var = jnp.mean((x - mu) ** 2, axis=0, keepdims=True)
    return (x - mu) / jnp.sqrt(var + BN_EPS)


def _dense1_body(x_ref, w1_ref, alb_ref, arb_ref, wres_ref,
                 ft_ref, atab_ref, res_ref):
    x = x_ref[...]
    bn = _bn_cols(x)
    ft = jnp.dot(bn, w1_ref[...], preferred_element_type=jnp.float32)
    a1 = jnp.dot(ft, alb_ref[...], preferred_element_type=jnp.float32)
    a2 = jnp.dot(ft, arb_ref[...], preferred_element_type=jnp.float32)
    res_ref[...] = jnp.dot(bn, wres_ref[...], preferred_element_type=jnp.float32)
    for h in range(HEADS1):
        ft_ref[h * N:(h + 1) * N, :] = ft[:, h * D_H1:(h + 1) * D_H1]
    atab_ref[:, 0, :] = jnp.transpose(a1)
    atab_ref[:, 1, :] = jnp.transpose(a2)


def _dense1(features, w1cat, alblk, arblk, wres1cat):
    return pl.pallas_call(
        _dense1_body,
        out_shape=[
            jax.ShapeDtypeStruct((HEADS1 * N, D_H1), jnp.float32),
            jax.ShapeDtypeStruct((HEADS1, 2, N), jnp.float32),
            jax.ShapeDtypeStruct((N, D1), jnp.float32),
        ],
    )(features, w1cat, alblk, arblk, wres1cat)


def _dense2_body(sc1a_ref, sc1b_ref, res1_ref, w2_ref, al2_ref, ar2_ref,
                 wres2_ref, ft2_ref, atab2_ref, res2_ref):
    cols = []
    for h in range(HEADS1):
        ref = (sc1a_ref, sc1b_ref)[h // 2]
        c = h % 2
        numer = ref[c * N:(c + 1) * N, 0:D_H1]
        denom = ref[c * N:(c + 1) * N, D_H1:D_H1 + 1]
        cols.append(numer / (denom + EPS_D))
    agg = jnp.concatenate(cols, axis=1)
    h1 = jnp.maximum(agg + res1_ref[...], 0.0)
    bn = _bn_cols(h1)
    ft2 = jnp.dot(bn, w2_ref[...], preferred_element_type=jnp.float32)
    b1 = jnp.dot(ft2, al2_ref[...], preferred_element_type=jnp.float32)
    b2 = jnp.dot(ft2, ar2_ref[...], preferred_element_type=jnp.float32)
    ft2_ref[...] = jnp.concatenate(
        [ft2, jnp.zeros((N, FT2W - D_OUT), jnp.float32)], axis=1)
    atab2_ref[0:1, :] = jnp.transpose(b1)
    atab2_ref[1:2, :] = jnp.transpose(b2)
    res2_ref[...] = jnp.dot(bn, wres2_ref[...], preferred_element_type=jnp.float32)


def _dense2(sc1a, sc1b, res1, w2, al2, ar2, wres2):
    return pl.pallas_call(
        _dense2_body,
        out_shape=[
            jax.ShapeDtypeStruct((N, FT2W), jnp.float32),
            jax.ShapeDtypeStruct((2, N), jnp.float32),
            jax.ShapeDtypeStruct((N, D_OUT), jnp.float32),
        ],
    )(sc1a, sc1b, res1, w2, al2, ar2, wres2)


def _combine_body(sc2_ref, res2_ref, out_ref):
    numer = sc2_ref[0:N, 0:D_OUT] + sc2_ref[N:2 * N, 0:D_OUT]
    denom = sc2_ref[0:N, D_OUT:D_OUT + 1] + sc2_ref[N:2 * N, D_OUT:D_OUT + 1]
    out_ref[...] = numer / (denom + EPS_D) + res2_ref[...]


def _combine(sc2, res2):
    return pl.pallas_call(
        _combine_body,
        out_shape=jax.ShapeDtypeStruct((N, D_OUT), jnp.float32),
    )(sc2, res2)


def _make_edge_kernel(dft, dreal, row, ch, ept, head_base, split_edges):
    nch = ept // ch
    zrows = 16
    mesh = plsc.VectorSubcoreMesh(core_axis_name="c", subcore_axis_name="s")
    nv = row // 16
    nf = dft // 16
    dv, dl = divmod(dreal, 16)

    @functools.partial(
        pl.kernel,
        mesh=mesh,
        compiler_params=_SC_PARAMS,
        out_type=jax.ShapeDtypeStruct((2 * N, row), jnp.float32),
        scratch_types=[
            pltpu.VMEM((ept,), jnp.int32),
            pltpu.VMEM((nch, ch), jnp.int32),
            pltpu.VMEM((2, N), jnp.float32),
            pltpu.VMEM((2, ch), jnp.int32),
            pltpu.VMEM((2, ch, dft), jnp.float32),
            pltpu.VMEM((ch, row), jnp.float32),
            pltpu.VMEM((zrows, row), jnp.float32),
            pltpu.VMEM_SHARED((N, row), jnp.float32),
            pltpu.SemaphoreType.DMA,
            pltpu.SemaphoreType.DMA,
            pltpu.SemaphoreType.DMA,
        ],
    )
    def edge_kernel(ft_hbm, atab_hbm, src_hbm, dstm_hbm, out_hbm,
                    src_v, dstm_v, atab_v, idxb, ftb, msg, zbuf,
                    accum, sem_a, sem_b, sem_c):
        c = lax.axis_index("c")
        s = lax.axis_index("s")
        if split_edges:
            ebase = (c * 16 + s) * ept
            widx = c * 16 + s
            toff = c * 0
        else:
            ebase = s * ept
            widx = s
            toff = (head_base + c) * N

        pltpu.sync_copy(src_hbm.at[pl.ds(ebase, ept)], src_v)
        pltpu.sync_copy(dstm_hbm.at[widx], dstm_v)
        if split_edges:
            pltpu.sync_copy(atab_hbm.at[0], atab_v)
        else:
            pltpu.sync_copy(atab_hbm.at[head_base + c], atab_v)

        rbase = s * 624
        nz = jnp.where(s == 15, 40, 39)

        def zr(i, carry):
            for j in range(nv):
                zbuf[i, pl.ds(j * 16, 16)] = jnp.zeros((16,), jnp.float32)
            return carry
        lax.fori_loop(0, zrows, zr, 0)

        def zc(k, carry):
            pltpu.sync_copy(zbuf, accum.at[pl.ds(rbase + k * zrows, zrows)])
            return carry
        lax.fori_loop(0, nz, zc, 0)
        plsc.subcore_barrier()

        iot = lax.iota(jnp.int32, 16)
        oh = (iot == dl).astype(jnp.float32)
        zi = jnp.zeros((16,), jnp.int32)
        oi = jnp.full((16,), 1, jnp.int32)

        def stage_idx(ci, b):
            cc = jnp.minimum(ci, nch - 1)
            for g in range(ch // 16):
                v = src_v[pl.ds(cc * ch + g * 16, 16)]
                idxb[b, pl.ds(g * 16, 16)] = v + toff

        def start(b, sem):
            pltpu.make_async_copy(ft_hbm.at[idxb.at[b]], ftb.at[b], sem).start()

        def wait(b, sem):
            pltpu.make_async_copy(ft_hbm.at[idxb.at[b]], ftb.at[b], sem).wait()

        def scat(ci):
            return pltpu.make_async_copy(msg, accum.at[dstm_v.at[ci]], sem_c)

        def process(ci, b):
            wv = []
            for g in range(ch // 16):
                sl = pl.ds(g * 16, 16)
                srcs = src_v[pl.ds(ci * ch + g * 16, 16)]
                dsts = dstm_v[ci, sl]
                av = plsc.load_gather(atab_v, [zi, srcs])
                bv = plsc.load_gather(atab_v, [oi, dsts])
                e = av + bv
                e = jnp.where(e >= 0.0, e, 0.01 * e)
                e = jnp.minimum(e, CLAMP)
                wv.append(jnp.exp(e))
            @pl.when(ci >= 1)
            def _():
                scat(ci).wait()
            for k in range(ch):
                w0 = wv[k // 16][k % 16]
                for j in range(nv):
                    if j < nf:
                        fv = ftb[b, k, pl.ds(j * 16, 16)]
                        if j == dv:
                            fv = fv + oh
                    else:
                        fv = oh
                    msg[k, pl.ds(j * 16, 16)] = fv * w0
            scat(ci).start(add=True)

        stage_idx(0, 0)
        start(0, sem_a)

        def step(i2, carry):
            i = i2 * 2
            wait(0, sem_a)
            stage_idx(i + 1, 1)
            start(1, sem_b)
            process(i, 0)
            wait(1, sem_b)
            stage_idx(i + 2, 0)
            start(0, sem_a)
            process(i + 1, 1)
            return carry
        lax.fori_loop(0, nch // 2, step, 0)
        if nch % 2 == 1:
            wait(0, sem_a)
            process(nch - 1, 0)
        else:
            wait(0, sem_a)
        scat(nch - 1).wait()

        plsc.subcore_barrier()

        def oc(k, carry):
            pltpu.sync_copy(accum.at[pl.ds(rbase + k * zrows, zrows)],
                            out_hbm.at[pl.ds(c * N + rbase + k * zrows, zrows)])
            return carry
        lax.fori_loop(0, nz, oc, 0)

    return edge_kernel


def kernel(features, edge_index, W1, al1, ar1, Wres1, W2, al2, ar2, Wres2):
    _edge1a = _make_edge_kernel(dft=64, dreal=64, row=ROW1, ch=80, ept=E // 16,
                                head_base=0, split_edges=False)
    _edge1b = _make_edge_kernel(dft=64, dreal=64, row=ROW1, ch=80, ept=E // 16,
                                head_base=2, split_edges=False)
    _edge2 = _make_edge_kernel(dft=FT2W, dreal=D_OUT, row=ROW2, ch=80,
                               ept=E // 32, split_edges=True, head_base=0)
    src = edge_index[0]
    dstm1 = edge_index[1].reshape(16, E // 16 // 80, 80)
    dstm2 = edge_index[1].reshape(32, E // 32 // 80, 80)

    w1cat = jnp.transpose(W1, (1, 0, 2)).reshape(D_IN, D1)
    wres1cat = jnp.transpose(Wres1, (1, 0, 2)).reshape(D_IN, D1)
    alblk = jnp.zeros((D1, HEADS1), jnp.float32)
    arblk = jnp.zeros((D1, HEADS1), jnp.float32)
    for h in range(HEADS1):
        alblk = alblk.at[h * D_H1:(h + 1) * D_H1, h].set(al1[h, :, 0])
        arblk = arblk.at[h * D_H1:(h + 1) * D_H1, h].set(ar1[h, :, 0])

    ft_tab, atab, res1 = _dense1(features, w1cat, alblk, arblk, wres1cat)
    sc1a = _edge1a(ft_tab, atab, src, dstm1)
    sc1b = _edge1b(ft_tab, atab, src, dstm1)
    ft2_tab, atab2, res2 = _dense2(sc1a, sc1b, res1, W2, al2, ar2, Wres2)
    sc2 = _edge2(ft2_tab, atab2.reshape(1, 2, N), src, dstm2)
    return _combine(sc2, res2)
